# Initial kernel scaffold; baseline (speedup 1.0000x reference)
import jax
import jax.numpy as jnp
from jax.experimental import pallas as pl

DIM = 1024
K = 4096
KR = 16
ETA = 0.1
DECAY = 0.01
T = 2048

BT = 256
BK = 512
NT = T // BT
NK = K // BK

_BF = jnp.bfloat16
_F32 = jnp.float32


def _qv_body(x_ref, qvw_ref, wm_ref, wmb_ref, q_ref, v_ref, m_ref):
    x = x_ref[...]
    qv = jax.lax.dot_general(x, qvw_ref[...], (((1,), (1,)), ((), ())),
                             preferred_element_type=_F32)
    q_ref[...] = qv[:, :DIM].astype(_BF)
    v_ref[...] = qv[:, DIM:]
    mlin = jax.lax.dot_general(x, wm_ref[...], (((1,), (1,)), ((), ())),
                               preferred_element_type=_F32)
    m_ref[...] = jax.nn.sigmoid(mlin + wmb_ref[...])


def _logits_body(q_ref, x_ref, s_ref, wnx_ref, wnm_ref, wnb_ref, m_ref,
                 temp_ref, rl_ref, wl_ref):
    s = s_ref[...]
    sf = s.astype(_F32)
    ss = jnp.sum(sf * sf, axis=1)
    scale = 1.0 / (jnp.sqrt(ss) + 1e-6)
    rl = jax.lax.dot_general(q_ref[...], s, (((1,), (1,)), ((), ())),
                             preferred_element_type=_F32)
    rl_ref[...] = rl * (scale[None, :] * temp_ref[...])
    wl = jax.lax.dot_general(x_ref[...], wnx_ref[...], (((1,), (1,)), ((), ())),
                             preferred_element_type=_F32)
    wl_ref[...] = wl + m_ref[...] * wnm_ref[0] + wnb_ref[0]


def _topk_body(l_ref, idx_ref, p_ref):
    l = l_ref[...]
    iota = jax.lax.broadcasted_iota(jnp.int32, (BT, K), 1)
    vals = []
    idxs = []
    for _ in range(KR):
        mx = jnp.max(l, axis=1, keepdims=True)
        cand = jnp.where(l == mx, iota, K)
        am = jnp.min(cand, axis=1, keepdims=True)
        vals.append(mx)
        idxs.append(am)
        l = jnp.where(iota == am, -jnp.inf, l)
    v = jnp.concatenate(vals, axis=1)
    i = jnp.concatenate(idxs, axis=1)
    e = jnp.exp(v - v[:, :1])
    idx_ref[...] = i
    p_ref[...] = e / jnp.sum(e, axis=1, keepdims=True)


def _route_tile(idx, p, kblk):
    kio = jax.lax.broadcasted_iota(jnp.int32, (BT, BK), 1) + kblk * BK
    tile = jnp.zeros((BT, BK), _F32)
    for j in range(KR):
        tile = tile + jnp.where(idx[:, j:j + 1] == kio, p[:, j:j + 1], 0.0)
    return tile.astype(_BF)


def _pre_body(idx_ref, p_ref, s_ref, acc_ref):
    k = pl.program_id(1)
    tile = _route_tile(idx_ref[...], p_ref[...], k)
    contrib = jnp.dot(tile, s_ref[...], preferred_element_type=_F32)

    @pl.when(k == 0)
    def _():
        acc_ref[...] = contrib

    @pl.when(k != 0)
    def _():
        acc_ref[...] = acc_ref[...] + contrib


def _d_body(idx_ref, p_ref, v_ref, pre_ref, d_ref):
    kblk = pl.program_id(0)
    t = pl.program_id(1)
    tile = _route_tile(idx_ref[...], p_ref[...], kblk)
    vo = ((v_ref[...] - pre_ref[...]) * ETA).astype(_BF)
    contrib = jax.lax.dot_general(tile, vo, (((0,), (0,)), ((), ())),
                                  preferred_element_type=_F32)

    @pl.when(t == 0)
    def _():
        d_ref[...] = contrib

    @pl.when(t != 0)
    def _():
        d_ref[...] = d_ref[...] + contrib


def _out_body(idx_ref, p_ref, d_ref, pre_ref, x_ref, o_ref):
    k = pl.program_id(1)
    tile = _route_tile(idx_ref[...], p_ref[...], k)
    contrib = jnp.dot(tile, d_ref[...], preferred_element_type=_F32)

    @pl.when(k == 0)
    def _():
        o_ref[...] = contrib

    @pl.when(k != 0)
    def _():
        o_ref[...] = o_ref[...] + contrib

    @pl.when(k == NK - 1)
    def _():
        o_ref[...] = o_ref[...] + (1.0 - DECAY) * pre_ref[...] + x_ref[...]


def _topk_call(logits):
    return pl.pallas_call(
        _topk_body,
        grid=(NT,),
        in_specs=[pl.BlockSpec((BT, K), lambda t: (t, 0))],
        out_specs=[pl.BlockSpec((BT, KR), lambda t: (t, 0)),
                   pl.BlockSpec((BT, KR), lambda t: (t, 0))],
        out_shape=[jax.ShapeDtypeStruct((T, KR), jnp.int32),
                   jax.ShapeDtypeStruct((T, KR), _F32)],
    )(logits)


def kernel(x, qkv_w, S_init, read_temperature, Wn_w, Wn_b, Wm_w, Wm_b):
    x2 = x.reshape(T, DIM)
    xb = x2.astype(_BF)
    qvw = jnp.concatenate([qkv_w[:DIM], qkv_w[2 * DIM:]], axis=0).astype(_BF)
    sb = S_init.astype(_BF)
    wnx = Wn_w[:, :DIM].astype(_BF)
    wnm = Wn_w[:, DIM].reshape(NK, 1, BK)
    wnb = Wn_b.reshape(NK, 1, BK)
    wmb = Wm_b.reshape(1, 1)
    wm = Wm_w.astype(_BF)
    temp = read_temperature.reshape(1, 1)

    q, v, m = pl.pallas_call(
        _qv_body,
        grid=(NT,),
        in_specs=[pl.BlockSpec((BT, DIM), lambda t: (t, 0)),
                  pl.BlockSpec((2 * DIM, DIM), lambda t: (0, 0)),
                  pl.BlockSpec((1, DIM), lambda t: (0, 0)),
                  pl.BlockSpec((1, 1), lambda t: (0, 0))],
        out_specs=[pl.BlockSpec((BT, DIM), lambda t: (t, 0)),
                   pl.BlockSpec((BT, DIM), lambda t: (t, 0)),
                   pl.BlockSpec((BT, 1), lambda t: (t, 0))],
        out_shape=[jax.ShapeDtypeStruct((T, DIM), _BF),
                   jax.ShapeDtypeStruct((T, DIM), _F32),
                   jax.ShapeDtypeStruct((T, 1), _F32)],
    )(xb, qvw, wm, wmb)

    rl, wl = pl.pallas_call(
        _logits_body,
        grid=(NT, NK),
        in_specs=[pl.BlockSpec((BT, DIM), lambda t, k: (t, 0)),
                  pl.BlockSpec((BT, DIM), lambda t, k: (t, 0)),
                  pl.BlockSpec((BK, DIM), lambda t, k: (k, 0)),
                  pl.BlockSpec((BK, DIM), lambda t, k: (k, 0)),
                  pl.BlockSpec((1, 1, BK), lambda t, k: (k, 0, 0)),
                  pl.BlockSpec((1, 1, BK), lambda t, k: (k, 0, 0)),
                  pl.BlockSpec((BT, 1), lambda t, k: (t, 0)),
                  pl.BlockSpec((1, 1), lambda t, k: (0, 0))],
        out_specs=[pl.BlockSpec((BT, BK), lambda t, k: (t, k)),
                   pl.BlockSpec((BT, BK), lambda t, k: (t, k))],
        out_shape=[jax.ShapeDtypeStruct((T, K), _F32),
                   jax.ShapeDtypeStruct((T, K), _F32)],
    )(q, xb, sb, wnx, wnm, wnb, m, temp)

    ridx, rp = _topk_call(rl)
    widx, wp = _topk_call(wl)

    pre = pl.pallas_call(
        _pre_body,
        grid=(NT, NK),
        in_specs=[pl.BlockSpec((BT, KR), lambda t, k: (t, 0)),
                  pl.BlockSpec((BT, KR), lambda t, k: (t, 0)),
                  pl.BlockSpec((BK, DIM), lambda t, k: (k, 0))],
        out_specs=pl.BlockSpec((BT, DIM), lambda t, k: (t, 0)),
        out_shape=jax.ShapeDtypeStruct((T, DIM), _F32),
    )(ridx, rp, sb)

    d = pl.pallas_call(
        _d_body,
        grid=(NK, NT),
        in_specs=[pl.BlockSpec((BT, KR), lambda k, t: (t, 0)),
                  pl.BlockSpec((BT, KR), lambda k, t: (t, 0)),
                  pl.BlockSpec((BT, DIM), lambda k, t: (t, 0)),
                  pl.BlockSpec((BT, DIM), lambda k, t: (t, 0))],
        out_specs=pl.BlockSpec((BK, DIM), lambda k, t: (k, 0)),
        out_shape=jax.ShapeDtypeStruct((K, DIM), _F32),
    )(widx, wp, v, pre)

    out = pl.pallas_call(
        _out_body,
        grid=(NT, NK),
        in_specs=[pl.BlockSpec((BT, KR), lambda t, k: (t, 0)),
                  pl.BlockSpec((BT, KR), lambda t, k: (t, 0)),
                  pl.BlockSpec((BK, DIM), lambda t, k: (k, 0)),
                  pl.BlockSpec((BT, DIM), lambda t, k: (t, 0)),
                  pl.BlockSpec((BT, DIM), lambda t, k: (t, 0))],
        out_specs=pl.BlockSpec((BT, DIM), lambda t, k: (t, 0)),
        out_shape=jax.ShapeDtypeStruct((T, DIM), _F32),
    )(ridx, rp, d.astype(_BF), pre, x2)

    return out.reshape(x.shape)


# speedup vs baseline: 5.4525x; 5.4525x over previous
import jax
import jax.numpy as jnp
from jax.experimental import pallas as pl

DIM = 1024
K = 4096
KR = 16
ETA = 0.1
DECAY = 0.01
T = 2048

BT = 256
BK = 512
NT = T // BT
NK = K // BK

_BF = jnp.bfloat16
_F32 = jnp.float32


def _qv_body(x_ref, qvw_ref, wm_ref, wmb_ref, q_ref, v_ref, m_ref):
    x = x_ref[...]
    qv = jax.lax.dot_general(x, qvw_ref[...], (((1,), (1,)), ((), ())),
                             preferred_element_type=_F32)
    q_ref[...] = qv[:, :DIM].astype(_BF)
    v_ref[...] = qv[:, DIM:]
    xw = x.astype(_F32) * wm_ref[...].astype(_F32)
    mlin = jnp.sum(xw, axis=1, keepdims=True)
    m_ref[...] = jax.nn.sigmoid(mlin + wmb_ref[...])


def _logits_body(q_ref, x_ref, s_ref, wnx_ref, wnm_ref, wnb_ref, m_ref,
                 temp_ref, rl_ref, wl_ref):
    s = s_ref[...]
    sf = s.astype(_F32)
    ss = jnp.sum(sf * sf, axis=1)
    scale = 1.0 / (jnp.sqrt(ss) + 1e-6)
    rl = jax.lax.dot_general(q_ref[...], s, (((1,), (1,)), ((), ())),
                             preferred_element_type=_F32)
    rl_ref[...] = rl * (scale[None, :] * temp_ref[...])
    wl = jax.lax.dot_general(x_ref[...], wnx_ref[...], (((1,), (1,)), ((), ())),
                             preferred_element_type=_F32)
    wl_ref[...] = wl + m_ref[...] * wnm_ref[0] + wnb_ref[0]


def _topk_body(l_ref, idx_ref, p_ref):
    l = l_ref[...]
    iota = jax.lax.broadcasted_iota(jnp.int32, (BT, K), 1)
    vals = []
    idxs = []
    for _ in range(KR):
        mx = jnp.max(l, axis=1, keepdims=True)
        cand = jnp.where(l == mx, iota, K)
        am = jnp.min(cand, axis=1, keepdims=True)
        vals.append(mx)
        idxs.append(am)
        l = jnp.where(iota == am, -jnp.inf, l)
    v = jnp.concatenate(vals, axis=1)
    i = jnp.concatenate(idxs, axis=1)
    e = jnp.exp(v - v[:, :1])
    idx_ref[...] = i
    p_ref[...] = e / jnp.sum(e, axis=1, keepdims=True)


def _route_tile(idx, p, kblk):
    kio = jax.lax.broadcasted_iota(jnp.int32, (BT, BK), 1) + kblk * BK
    tile = jnp.zeros((BT, BK), _F32)
    for j in range(KR):
        tile = tile + jnp.where(idx[:, j:j + 1] == kio, p[:, j:j + 1], 0.0)
    return tile.astype(_BF)


def _pre_body(idx_ref, p_ref, s_ref, acc_ref):
    k = pl.program_id(1)
    tile = _route_tile(idx_ref[...], p_ref[...], k)
    contrib = jnp.dot(tile, s_ref[...], preferred_element_type=_F32)

    @pl.when(k == 0)
    def _():
        acc_ref[...] = contrib

    @pl.when(k != 0)
    def _():
        acc_ref[...] = acc_ref[...] + contrib


def _d_body(idx_ref, p_ref, v_ref, pre_ref, d_ref):
    kblk = pl.program_id(0)
    t = pl.program_id(1)
    tile = _route_tile(idx_ref[...], p_ref[...], kblk)
    vo = ((v_ref[...] - pre_ref[...]) * ETA).astype(_BF)
    contrib = jax.lax.dot_general(tile, vo, (((0,), (0,)), ((), ())),
                                  preferred_element_type=_F32)

    @pl.when(t == 0)
    def _():
        d_ref[...] = contrib

    @pl.when(t != 0)
    def _():
        d_ref[...] = d_ref[...] + contrib


def _out_body(idx_ref, p_ref, d_ref, pre_ref, x_ref, o_ref):
    k = pl.program_id(1)
    tile = _route_tile(idx_ref[...], p_ref[...], k)
    contrib = jnp.dot(tile, d_ref[...], preferred_element_type=_F32)

    @pl.when(k == 0)
    def _():
        o_ref[...] = contrib

    @pl.when(k != 0)
    def _():
        o_ref[...] = o_ref[...] + contrib

    @pl.when(k == NK - 1)
    def _():
        o_ref[...] = o_ref[...] + (1.0 - DECAY) * pre_ref[...] + x_ref[...]


def _topk_call(logits):
    return pl.pallas_call(
        _topk_body,
        grid=(NT,),
        in_specs=[pl.BlockSpec((BT, K), lambda t: (t, 0))],
        out_specs=[pl.BlockSpec((BT, KR), lambda t: (t, 0)),
                   pl.BlockSpec((BT, KR), lambda t: (t, 0))],
        out_shape=[jax.ShapeDtypeStruct((T, KR), jnp.int32),
                   jax.ShapeDtypeStruct((T, KR), _F32)],
    )(logits)


def kernel(x, qkv_w, S_init, read_temperature, Wn_w, Wn_b, Wm_w, Wm_b):
    x2 = x.reshape(T, DIM)
    xb = x2.astype(_BF)
    qvw = jnp.concatenate([qkv_w[:DIM], qkv_w[2 * DIM:]], axis=0).astype(_BF)
    sb = S_init.astype(_BF)
    wnx = Wn_w[:, :DIM].astype(_BF)
    wnm = Wn_w[:, DIM].reshape(NK, 1, BK)
    wnb = Wn_b.reshape(NK, 1, BK)
    wmb = Wm_b.reshape(1, 1)
    wm = Wm_w.astype(_BF)
    temp = read_temperature.reshape(1, 1)

    q, v, m = pl.pallas_call(
        _qv_body,
        grid=(NT,),
        in_specs=[pl.BlockSpec((BT, DIM), lambda t: (t, 0)),
                  pl.BlockSpec((2 * DIM, DIM), lambda t: (0, 0)),
                  pl.BlockSpec((1, DIM), lambda t: (0, 0)),
                  pl.BlockSpec((1, 1), lambda t: (0, 0))],
        out_specs=[pl.BlockSpec((BT, DIM), lambda t: (t, 0)),
                   pl.BlockSpec((BT, DIM), lambda t: (t, 0)),
                   pl.BlockSpec((BT, 1), lambda t: (t, 0))],
        out_shape=[jax.ShapeDtypeStruct((T, DIM), _BF),
                   jax.ShapeDtypeStruct((T, DIM), _F32),
                   jax.ShapeDtypeStruct((T, 1), _F32)],
    )(xb, qvw, wm, wmb)

    rl, wl = pl.pallas_call(
        _logits_body,
        grid=(NT, NK),
        in_specs=[pl.BlockSpec((BT, DIM), lambda t, k: (t, 0)),
                  pl.BlockSpec((BT, DIM), lambda t, k: (t, 0)),
                  pl.BlockSpec((BK, DIM), lambda t, k: (k, 0)),
                  pl.BlockSpec((BK, DIM), lambda t, k: (k, 0)),
                  pl.BlockSpec((1, 1, BK), lambda t, k: (k, 0, 0)),
                  pl.BlockSpec((1, 1, BK), lambda t, k: (k, 0, 0)),
                  pl.BlockSpec((BT, 1), lambda t, k: (t, 0)),
                  pl.BlockSpec((1, 1), lambda t, k: (0, 0))],
        out_specs=[pl.BlockSpec((BT, BK), lambda t, k: (t, k)),
                   pl.BlockSpec((BT, BK), lambda t, k: (t, k))],
        out_shape=[jax.ShapeDtypeStruct((T, K), _F32),
                   jax.ShapeDtypeStruct((T, K), _F32)],
    )(q, xb, sb, wnx, wnm, wnb, m, temp)

    ridx, rp = _topk_call(rl)
    widx, wp = _topk_call(wl)

    pre = pl.pallas_call(
        _pre_body,
        grid=(NT, NK),
        in_specs=[pl.BlockSpec((BT, KR), lambda t, k: (t, 0)),
                  pl.BlockSpec((BT, KR), lambda t, k: (t, 0)),
                  pl.BlockSpec((BK, DIM), lambda t, k: (k, 0))],
        out_specs=pl.BlockSpec((BT, DIM), lambda t, k: (t, 0)),
        out_shape=jax.ShapeDtypeStruct((T, DIM), _F32),
    )(ridx, rp, sb)

    d = pl.pallas_call(
        _d_body,
        grid=(NK, NT),
        in_specs=[pl.BlockSpec((BT, KR), lambda k, t: (t, 0)),
                  pl.BlockSpec((BT, KR), lambda k, t: (t, 0)),
                  pl.BlockSpec((BT, DIM), lambda k, t: (t, 0)),
                  pl.BlockSpec((BT, DIM), lambda k, t: (t, 0))],
        out_specs=pl.BlockSpec((BK, DIM), lambda k, t: (k, 0)),
        out_shape=jax.ShapeDtypeStruct((K, DIM), _F32),
    )(widx, wp, v, pre)

    out = pl.pallas_call(
        _out_body,
        grid=(NT, NK),
        in_specs=[pl.BlockSpec((BT, KR), lambda t, k: (t, 0)),
                  pl.BlockSpec((BT, KR), lambda t, k: (t, 0)),
                  pl.BlockSpec((BK, DIM), lambda t, k: (k, 0)),
                  pl.BlockSpec((BT, DIM), lambda t, k: (t, 0)),
                  pl.BlockSpec((BT, DIM), lambda t, k: (t, 0))],
        out_specs=pl.BlockSpec((BT, DIM), lambda t, k: (t, 0)),
        out_shape=jax.ShapeDtypeStruct((T, DIM), _F32),
    )(ridx, rp, d.astype(_BF), pre, x2)

    return out.reshape(x.shape)
